# baseline (speedup 1.0000x reference)
import jax
import jax.numpy as jnp
from jax.experimental import pallas as pl
from jax.experimental.pallas import tpu as pltpu

_M = 4096
_N = 4096
_BM = 256


def _loss_body(A_ref, b_ref, x_ref, Iy_ref, il_ref, iu_ref, l_ref, u_ref,
               out_ref, acc_ref):
    i = pl.program_id(0)
    nb = pl.num_programs(0)

    @pl.when(i == 0)
    def _init():
        xv = x_ref[...]
        vv = (jnp.maximum(l_ref[...] - xv, 0.0) * il_ref[...]
              + jnp.maximum(xv - u_ref[...], 0.0) * iu_ref[...])
        bv = b_ref[...]
        acc_ref[0] = jnp.sum(vv * vv)
        acc_ref[1] = jnp.sum(bv * bv)
        acc_ref[2] = 0.0

    xt = x_ref[...].reshape(1, _N)
    ax = jnp.sum(A_ref[...] * xt, axis=1, keepdims=True)
    bb = b_ref[pl.ds(i * _BM, _BM), :]
    cv = bb - ax
    cv = cv + jnp.maximum(-cv, 0.0) * Iy_ref[pl.ds(i * _BM, _BM), :]
    acc_ref[2] += jnp.sum(cv * cv)

    @pl.when(i == nb - 1)
    def _fin():
        part_2 = jnp.sqrt(acc_ref[0] + acc_ref[2])
        part_3 = 1.0 + jnp.sqrt(acc_ref[1])
        out_ref[0] = part_2 / part_3


def kernel(A, b, c, x, Iy, il, iu, l, u):
    del c
    b2 = b.reshape(_M, 1)
    full_m = pl.BlockSpec((_M, 1), lambda i: (0, 0))
    full_n = pl.BlockSpec((_N, 1), lambda i: (0, 0))
    out = pl.pallas_call(
        _loss_body,
        grid=(_M // _BM,),
        in_specs=[
            pl.BlockSpec((_BM, _N), lambda i: (i, 0)),
            full_m,
            full_n,
            full_m,
            full_n,
            full_n,
            full_n,
            full_n,
        ],
        out_specs=pl.BlockSpec(memory_space=pltpu.SMEM),
        out_shape=jax.ShapeDtypeStruct((1,), jnp.float32),
        scratch_shapes=[pltpu.SMEM((3,), jnp.float32)],
    )(A, b2, x, Iy, il, iu, l, u)
    return out[0]
